# Initial kernel scaffold; baseline (speedup 1.0000x reference)
import functools

import jax
import jax.numpy as jnp
from jax import lax
from jax.experimental import pallas as pl
from jax.experimental.pallas import tpu as pltpu
from jax.experimental.pallas import tpu_sc as plsc

H = 512
D = 128
NT = 8000
NV = 2000
NN = 10000
E = 320000

BM = 400


def _encoder(x, W1, b1, g1, be1, W2, b2, g2, be2, W3, b3):
    n = x.shape[0]
    nb = n // BM
    inv_n = 1.0 / n

    def body(x_ref, W1_ref, b1_ref, g1_ref, be1_ref, W2_ref, b2_ref,
             g2_ref, be2_ref, W3_ref, b3_ref, out_ref, s1_ref, s2_ref, h2_ref):
        ph = pl.program_id(0)
        b = pl.program_id(1)
        xb = x_ref[...]
        h1 = jnp.dot(xb, W1_ref[...], preferred_element_type=jnp.float32) + b1_ref[...]

        @pl.when(ph == 0)
        def _():
            @pl.when(b == 0)
            def _():
                s1_ref[...] = jnp.zeros_like(s1_ref)
            s1_ref[...] += jnp.concatenate(
                [jnp.sum(h1, axis=0, keepdims=True),
                 jnp.sum(h1 * h1, axis=0, keepdims=True)], axis=0)

        @pl.when(ph == 1)
        def _():
            mu = s1_ref[0:1, :] * inv_n
            var = s1_ref[1:2, :] * inv_n - mu * mu
            hn = (h1 - mu) * lax.rsqrt(var + 1e-5) * g1_ref[...] + be1_ref[...]
            hr = jnp.maximum(hn, 0.0)
            h2 = jnp.dot(hr, W2_ref[...], preferred_element_type=jnp.float32) + b2_ref[...]
            h2_ref[pl.ds(b * BM, BM), :] = h2
            @pl.when(b == 0)
            def _():
                s2_ref[...] = jnp.zeros_like(s2_ref)
            s2_ref[...] += jnp.concatenate(
                [jnp.sum(h2, axis=0, keepdims=True),
                 jnp.sum(h2 * h2, axis=0, keepdims=True)], axis=0)

        @pl.when(ph == 2)
        def _():
            h2 = h2_ref[pl.ds(b * BM, BM), :]
            mu = s2_ref[0:1, :] * inv_n
            var = s2_ref[1:2, :] * inv_n - mu * mu
            hn = (h2 - mu) * lax.rsqrt(var + 1e-5) * g2_ref[...] + be2_ref[...]
            hr = jnp.maximum(hn, 0.0)
            out_ref[...] = jnp.dot(hr, W3_ref[...], preferred_element_type=jnp.float32) + b3_ref[...]

    cst = lambda p, b: (0, 0)
    return pl.pallas_call(
        body,
        grid=(3, nb),
        in_specs=[
            pl.BlockSpec((BM, 3), lambda p, b: (b, 0)),
            pl.BlockSpec((3, H), cst),
            pl.BlockSpec((1, H), cst),
            pl.BlockSpec((1, H), cst),
            pl.BlockSpec((1, H), cst),
            pl.BlockSpec((H, H), cst),
            pl.BlockSpec((1, H), cst),
            pl.BlockSpec((1, H), cst),
            pl.BlockSpec((1, H), cst),
            pl.BlockSpec((H, D), cst),
            pl.BlockSpec((1, D), cst),
        ],
        out_specs=pl.BlockSpec((BM, D), lambda p, b: (b, 0)),
        out_shape=jax.ShapeDtypeStruct((n, D), jnp.float32),
        scratch_shapes=[
            pltpu.VMEM((2, H), jnp.float32),
            pltpu.VMEM((2, H), jnp.float32),
            pltpu.VMEM((n, H), jnp.float32),
        ],
    )(x, W1, b1.reshape(1, H), g1.reshape(1, H), be1.reshape(1, H),
      W2, b2.reshape(1, H), g2.reshape(1, H), be2.reshape(1, H),
      W3, b3.reshape(1, D))


def _sc_agg(x_ch, src, dst):
    nc = x_ch.shape[0]
    BE = 128
    per_sub = E // 32
    nfull = per_sub // BE
    tail = per_sub - nfull * BE
    rps = NN // 16
    zr = 125
    mesh = plsc.VectorSubcoreMesh(core_axis_name="c", subcore_axis_name="s")

    @functools.partial(
        pl.kernel,
        out_type=jax.ShapeDtypeStruct((2, nc, NN, 128), jnp.float32),
        mesh=mesh,
        scratch_types=[
            pltpu.VMEM((BE,), jnp.int32),
            pltpu.VMEM((BE,), jnp.int32),
            pltpu.VMEM((BE, 128), jnp.float32),
            pltpu.VMEM((tail,), jnp.int32),
            pltpu.VMEM((tail,), jnp.int32),
            pltpu.VMEM((tail, 128), jnp.float32),
            pltpu.VMEM((zr, 128), jnp.float32),
            pltpu.VMEM_SHARED((NN, 128), jnp.float32),
            pltpu.SemaphoreType.DMA,
        ],
    )
    def k(x_hbm, src_hbm, dst_hbm, out_hbm,
          si_v, di_v, rows_v, sit_v, dit_v, rowst_v, zero_v, acc_sh, sem):
        cid = lax.axis_index("c")
        sid = lax.axis_index("s")
        base0 = cid * (E // 2) + sid * per_sub

        def zb(r, carry):
            for c in range(8):
                zero_v[r, pl.ds(c * 16, 16)] = jnp.zeros((16,), jnp.float32)
            return carry
        lax.fori_loop(0, zr, zb, 0)

        for ch in range(nc):
            for z in range(rps // zr):
                pltpu.sync_copy(zero_v, acc_sh.at[pl.ds(sid * rps + z * zr, zr)])
            plsc.subcore_barrier()

            def blk(i, carry):
                eb = base0 + i * BE
                pltpu.sync_copy(src_hbm.at[pl.ds(eb, BE)], si_v)
                pltpu.sync_copy(dst_hbm.at[pl.ds(eb, BE)], di_v)
                pltpu.async_copy(x_hbm.at[ch].at[si_v], rows_v, sem).wait()
                pltpu.sync_copy(rows_v, acc_sh.at[di_v], add=True)
                return carry
            lax.fori_loop(0, nfull, blk, 0)

            ebt = base0 + nfull * BE
            pltpu.sync_copy(src_hbm.at[pl.ds(ebt, tail)], sit_v)
            pltpu.sync_copy(dst_hbm.at[pl.ds(ebt, tail)], dit_v)
            pltpu.async_copy(x_hbm.at[ch].at[sit_v], rowst_v, sem).wait()
            pltpu.sync_copy(rowst_v, acc_sh.at[dit_v], add=True)

            plsc.subcore_barrier()
            pltpu.sync_copy(acc_sh.at[pl.ds(sid * rps, rps)],
                            out_hbm.at[cid, ch, pl.ds(sid * rps, rps)])

    return k(x_ch, src, dst)


def _gin_mlp(x_ch, agg, W1, b1, W2, b2, relu_out):
    ncx = x_ch.shape[0]
    H1 = W1.shape[1]
    Dout = W2.shape[1]
    ncy = Dout // 128
    nb = NN // BM
    cst = lambda b: (0, 0)

    def body(x_ref, agg_ref, W1_ref, b1_ref, W2_ref, b2_ref, out_ref):
        W1v = W1_ref[...]
        acc = jnp.zeros((BM, H1), jnp.float32)
        for c in range(ncx):
            t = x_ref[c] + agg_ref[0, c] + agg_ref[1, c]
            acc = acc + jnp.dot(t, W1v[c * 128:(c + 1) * 128, :],
                                preferred_element_type=jnp.float32)
        h = jnp.maximum(acc + b1_ref[...], 0.0)
        o = jnp.dot(h, W2_ref[...], preferred_element_type=jnp.float32) + b2_ref[...]
        if relu_out:
            o = jnp.maximum(o, 0.0)
        for c in range(ncy):
            out_ref[c] = o[:, c * 128:(c + 1) * 128]

    return pl.pallas_call(
        body,
        grid=(nb,),
        in_specs=[
            pl.BlockSpec((ncx, BM, 128), lambda b: (0, b, 0)),
            pl.BlockSpec((2, ncx, BM, 128), lambda b: (0, 0, b, 0)),
            pl.BlockSpec((ncx * 128, H1), cst),
            pl.BlockSpec((1, H1), cst),
            pl.BlockSpec((H1, Dout), cst),
            pl.BlockSpec((1, Dout), cst),
        ],
        out_specs=pl.BlockSpec((ncy, BM, 128), lambda b: (0, b, 0)),
        out_shape=jax.ShapeDtypeStruct((ncy, NN, 128), jnp.float32),
    )(x_ch, agg, W1, b1.reshape(1, H1), W2, b2.reshape(1, Dout))


def _gin_mlp_final(x_ch, agg, W1, b1, W2, b2):
    ncx = x_ch.shape[0]
    H1 = W1.shape[1]
    Dout = W2.shape[1]
    nb = NN // BM
    cst = lambda b: (0, 0)

    def body(x_ref, agg_ref, W1_ref, b1_ref, W2_ref, b2_ref,
             out_ref, g_ref, gacc_ref):
        b = pl.program_id(0)
        W1v = W1_ref[...]
        acc = jnp.zeros((BM, H1), jnp.float32)
        for c in range(ncx):
            t = x_ref[c] + agg_ref[0, c] + agg_ref[1, c]
            acc = acc + jnp.dot(t, W1v[c * 128:(c + 1) * 128, :],
                                preferred_element_type=jnp.float32)
        h = jnp.maximum(acc + b1_ref[...], 0.0)
        o = jnp.dot(h, W2_ref[...], preferred_element_type=jnp.float32) + b2_ref[...]
        out_ref[...] = o

        @pl.when(b == 0)
        def _():
            gacc_ref[...] = jnp.zeros_like(gacc_ref)
        gacc_ref[...] += jnp.sum(o, axis=0, keepdims=True)

        @pl.when(b == nb - 1)
        def _():
            g_ref[...] = gacc_ref[...] * (1.0 / NN)

    return pl.pallas_call(
        body,
        grid=(nb,),
        in_specs=[
            pl.BlockSpec((ncx, BM, 128), lambda b: (0, b, 0)),
            pl.BlockSpec((2, ncx, BM, 128), lambda b: (0, 0, b, 0)),
            pl.BlockSpec((ncx * 128, H1), cst),
            pl.BlockSpec((1, H1), cst),
            pl.BlockSpec((H1, Dout), cst),
            pl.BlockSpec((1, Dout), cst),
        ],
        out_specs=[
            pl.BlockSpec((BM, Dout), lambda b: (b, 0)),
            pl.BlockSpec((1, Dout), cst),
        ],
        out_shape=[
            jax.ShapeDtypeStruct((NN, Dout), jnp.float32),
            jax.ShapeDtypeStruct((1, Dout), jnp.float32),
        ],
        scratch_shapes=[pltpu.VMEM((1, Dout), jnp.float32)],
    )(x_ch, agg, W1, b1.reshape(1, H1), W2, b2.reshape(1, Dout))


def _sc_edge(ne, src, dst):
    BE = 128
    per_w = E // 32
    nfull = per_w // BE
    tail = per_w - nfull * BE
    mesh = plsc.VectorSubcoreMesh(core_axis_name="c", subcore_axis_name="s")

    @functools.partial(
        pl.kernel,
        out_type=jax.ShapeDtypeStruct((E, 256), jnp.float32),
        mesh=mesh,
        scratch_types=[
            pltpu.VMEM((BE,), jnp.int32),
            pltpu.VMEM((BE,), jnp.int32),
            pltpu.VMEM((BE, 128), jnp.float32),
            pltpu.VMEM((BE, 128), jnp.float32),
            pltpu.VMEM((tail,), jnp.int32),
            pltpu.VMEM((tail,), jnp.int32),
            pltpu.VMEM((tail, 128), jnp.float32),
            pltpu.VMEM((tail, 128), jnp.float32),
            pltpu.SemaphoreType.DMA,
            pltpu.SemaphoreType.DMA,
        ],
    )
    def k(ne_hbm, src_hbm, dst_hbm, out_hbm,
          si_v, di_v, rs_v, rd_v, sit_v, dit_v, rst_v, rdt_v, sem1, sem2):
        cid = lax.axis_index("c")
        sid = lax.axis_index("s")
        base0 = (sid * 2 + cid) * per_w

        def blk(i, carry):
            eb = base0 + i * BE
            pltpu.sync_copy(src_hbm.at[pl.ds(eb, BE)], si_v)
            pltpu.sync_copy(dst_hbm.at[pl.ds(eb, BE)], di_v)
            c1 = pltpu.async_copy(ne_hbm.at[si_v], rs_v, sem1)
            c2 = pltpu.async_copy(ne_hbm.at[di_v], rd_v, sem2)
            c1.wait()
            c2.wait()
            pltpu.sync_copy(rs_v, out_hbm.at[pl.ds(eb, BE), pl.ds(0, 128)])
            pltpu.sync_copy(rd_v, out_hbm.at[pl.ds(eb, BE), pl.ds(128, 128)])
            return carry
        lax.fori_loop(0, nfull, blk, 0)

        ebt = base0 + nfull * BE
        pltpu.sync_copy(src_hbm.at[pl.ds(ebt, tail)], sit_v)
        pltpu.sync_copy(dst_hbm.at[pl.ds(ebt, tail)], dit_v)
        c1 = pltpu.async_copy(ne_hbm.at[sit_v], rst_v, sem1)
        c2 = pltpu.async_copy(ne_hbm.at[dit_v], rdt_v, sem2)
        c1.wait()
        c2.wait()
        pltpu.sync_copy(rst_v, out_hbm.at[pl.ds(ebt, tail), pl.ds(0, 128)])
        pltpu.sync_copy(rdt_v, out_hbm.at[pl.ds(ebt, tail), pl.ds(128, 128)])

    return k(ne, src, dst)


def kernel(task_state_scheduled, task_state_ready, task_lengths,
           vm_completion_times, vm_speeds, vm_energy_rates, params,
           task_assignments, compatibilities, task_dependencies):
    p = params
    task_x = jnp.stack([task_state_scheduled, task_state_ready, task_lengths], axis=-1)
    vm_x = jnp.stack([vm_completion_times, vm_speeds, vm_energy_rates], axis=-1)

    task_h = _encoder(task_x, p['task_W1'], p['task_b1'], p['task_g1'], p['task_be1'],
                      p['task_W2'], p['task_b2'], p['task_g2'], p['task_be2'],
                      p['task_W3'], p['task_b3'])
    vm_h = _encoder(vm_x, p['vm_W1'], p['vm_b1'], p['vm_g1'], p['vm_be1'],
                    p['vm_W2'], p['vm_b2'], p['vm_g2'], p['vm_be2'],
                    p['vm_W3'], p['vm_b3'])
    node_x = jnp.concatenate([task_h, vm_h], axis=0)

    src = jnp.concatenate([compatibilities[0], task_dependencies[0]])
    dst = jnp.concatenate([compatibilities[1] + NT, task_dependencies[1]])

    x1 = node_x.reshape(1, NN, 128)
    agg1 = _sc_agg(x1, src, dst)
    h1 = _gin_mlp(x1, agg1, p['gin_W1a'], p['gin_b1a'], p['gin_W1b'], p['gin_b1b'], True)
    agg2 = _sc_agg(h1, src, dst)
    h2 = _gin_mlp(h1, agg2, p['gin_W2a'], p['gin_b2a'], p['gin_W2b'], p['gin_b2b'], True)
    agg3 = _sc_agg(h2, src, dst)
    node_emb, graph_emb = _gin_mlp_final(h2, agg3, p['gin_W3a'], p['gin_b3a'],
                                         p['gin_W3b'], p['gin_b3b'])
    edge_emb = _sc_edge(node_emb, src, dst)
    return node_emb, edge_emb, graph_emb


# speedup vs baseline: 3.9356x; 3.9356x over previous
import functools

import jax
import jax.numpy as jnp
from jax import lax
from jax.experimental import pallas as pl
from jax.experimental.pallas import tpu as pltpu
from jax.experimental.pallas import tpu_sc as plsc

H = 512
D = 128
NT = 8000
NV = 2000
NN = 10000
E = 320000

BM = 400


def _encoder(x, W1, b1, g1, be1, W2, b2, g2, be2, W3, b3):
    n = x.shape[0]
    nb = n // BM
    inv_n = 1.0 / n

    def body(x_ref, W1_ref, b1_ref, g1_ref, be1_ref, W2_ref, b2_ref,
             g2_ref, be2_ref, W3_ref, b3_ref, out_ref, s1_ref, s2_ref, h2_ref):
        ph = pl.program_id(0)
        b = pl.program_id(1)
        xb = x_ref[...]
        h1 = jnp.dot(xb, W1_ref[...], preferred_element_type=jnp.float32) + b1_ref[...]

        @pl.when(ph == 0)
        def _():
            @pl.when(b == 0)
            def _():
                s1_ref[...] = jnp.zeros_like(s1_ref)
            s1_ref[...] += jnp.concatenate(
                [jnp.sum(h1, axis=0, keepdims=True),
                 jnp.sum(h1 * h1, axis=0, keepdims=True)], axis=0)

        @pl.when(ph == 1)
        def _():
            mu = s1_ref[0:1, :] * inv_n
            var = s1_ref[1:2, :] * inv_n - mu * mu
            hn = (h1 - mu) * lax.rsqrt(var + 1e-5) * g1_ref[...] + be1_ref[...]
            hr = jnp.maximum(hn, 0.0)
            h2 = jnp.dot(hr, W2_ref[...], preferred_element_type=jnp.float32) + b2_ref[...]
            h2_ref[pl.ds(b * BM, BM), :] = h2
            @pl.when(b == 0)
            def _():
                s2_ref[...] = jnp.zeros_like(s2_ref)
            s2_ref[...] += jnp.concatenate(
                [jnp.sum(h2, axis=0, keepdims=True),
                 jnp.sum(h2 * h2, axis=0, keepdims=True)], axis=0)

        @pl.when(ph == 2)
        def _():
            h2 = h2_ref[pl.ds(b * BM, BM), :]
            mu = s2_ref[0:1, :] * inv_n
            var = s2_ref[1:2, :] * inv_n - mu * mu
            hn = (h2 - mu) * lax.rsqrt(var + 1e-5) * g2_ref[...] + be2_ref[...]
            hr = jnp.maximum(hn, 0.0)
            out_ref[...] = jnp.dot(hr, W3_ref[...], preferred_element_type=jnp.float32) + b3_ref[...]

    cst = lambda p, b: (0, 0)
    return pl.pallas_call(
        body,
        grid=(3, nb),
        in_specs=[
            pl.BlockSpec((BM, 3), lambda p, b: (b, 0)),
            pl.BlockSpec((3, H), cst),
            pl.BlockSpec((1, H), cst),
            pl.BlockSpec((1, H), cst),
            pl.BlockSpec((1, H), cst),
            pl.BlockSpec((H, H), cst),
            pl.BlockSpec((1, H), cst),
            pl.BlockSpec((1, H), cst),
            pl.BlockSpec((1, H), cst),
            pl.BlockSpec((H, D), cst),
            pl.BlockSpec((1, D), cst),
        ],
        out_specs=pl.BlockSpec((BM, D), lambda p, b: (b, 0)),
        out_shape=jax.ShapeDtypeStruct((n, D), jnp.float32),
        scratch_shapes=[
            pltpu.VMEM((2, H), jnp.float32),
            pltpu.VMEM((2, H), jnp.float32),
            pltpu.VMEM((n, H), jnp.float32),
        ],
    )(x, W1, b1.reshape(1, H), g1.reshape(1, H), be1.reshape(1, H),
      W2, b2.reshape(1, H), g2.reshape(1, H), be2.reshape(1, H),
      W3, b3.reshape(1, D))


def _sc_agg(x_ch, src, dst):
    nc = x_ch.shape[0]
    BE = 128
    per_sub = E // 32
    nfull = per_sub // BE
    tail = per_sub - nfull * BE
    rps = 624
    rtail = NN - 16 * rps
    mesh = plsc.VectorSubcoreMesh(core_axis_name="c", subcore_axis_name="s")

    @functools.partial(
        pl.kernel,
        out_type=jax.ShapeDtypeStruct((2, nc, NN, 128), jnp.float32),
        mesh=mesh,
        scratch_types=[
            pltpu.VMEM((BE,), jnp.int32),
            pltpu.VMEM((BE,), jnp.int32),
            pltpu.VMEM((BE, 128), jnp.float32),
            pltpu.VMEM((tail,), jnp.int32),
            pltpu.VMEM((tail,), jnp.int32),
            pltpu.VMEM((tail, 128), jnp.float32),
            pltpu.VMEM((16, 128), jnp.float32),
            pltpu.VMEM_SHARED((NN, 128), jnp.float32),
            pltpu.SemaphoreType.DMA,
        ],
    )
    def k(x_hbm, src_hbm, dst_hbm, out_hbm,
          si_v, di_v, rows_v, sit_v, dit_v, rowst_v, zero_v, acc_sh, sem):
        cid = lax.axis_index("c")
        sid = lax.axis_index("s")
        base0 = cid * (E // 2) + sid * per_sub

        def zb(r, carry):
            for c in range(8):
                zero_v[r, pl.ds(c * 16, 16)] = jnp.zeros((16,), jnp.float32)
            return carry
        lax.fori_loop(0, 16, zb, 0)

        for ch in range(nc):
            def zc(z, carry):
                pltpu.sync_copy(zero_v, acc_sh.at[pl.ds(sid * rps + z * 16, 16)])
                return carry
            lax.fori_loop(0, rps // 16, zc, 0)

            @pl.when(sid == 15)
            def _():
                pltpu.sync_copy(zero_v, acc_sh.at[pl.ds(16 * rps, rtail)])
            plsc.subcore_barrier()

            def blk(i, carry):
                eb = base0 + i * BE
                pltpu.sync_copy(src_hbm.at[pl.ds(eb, BE)], si_v)
                pltpu.sync_copy(dst_hbm.at[pl.ds(eb, BE)], di_v)
                pltpu.async_copy(x_hbm.at[ch].at[si_v], rows_v, sem).wait()
                pltpu.sync_copy(rows_v, acc_sh.at[di_v], add=True)
                return carry
            lax.fori_loop(0, nfull, blk, 0)

            ebt = base0 + nfull * BE
            pltpu.sync_copy(src_hbm.at[pl.ds(ebt, tail)], sit_v)
            pltpu.sync_copy(dst_hbm.at[pl.ds(ebt, tail)], dit_v)
            pltpu.async_copy(x_hbm.at[ch].at[sit_v], rowst_v, sem).wait()
            pltpu.sync_copy(rowst_v, acc_sh.at[dit_v], add=True)

            plsc.subcore_barrier()
            pltpu.sync_copy(acc_sh.at[pl.ds(sid * rps, rps)],
                            out_hbm.at[cid, ch, pl.ds(sid * rps, rps)])

            @pl.when(sid == 15)
            def _():
                pltpu.sync_copy(acc_sh.at[pl.ds(16 * rps, rtail)],
                                out_hbm.at[cid, ch, pl.ds(16 * rps, rtail)])

    return k(x_ch, src, dst)


def _gin_mlp(x_ch, agg, W1, b1, W2, b2, relu_out):
    ncx = x_ch.shape[0]
    H1 = W1.shape[1]
    Dout = W2.shape[1]
    ncy = Dout // 128
    nb = NN // BM
    cst = lambda b: (0, 0)

    def body(x_ref, agg_ref, W1_ref, b1_ref, W2_ref, b2_ref, out_ref):
        W1v = W1_ref[...]
        acc = jnp.zeros((BM, H1), jnp.float32)
        for c in range(ncx):
            t = x_ref[c] + agg_ref[0, c] + agg_ref[1, c]
            acc = acc + jnp.dot(t, W1v[c * 128:(c + 1) * 128, :],
                                preferred_element_type=jnp.float32)
        h = jnp.maximum(acc + b1_ref[...], 0.0)
        o = jnp.dot(h, W2_ref[...], preferred_element_type=jnp.float32) + b2_ref[...]
        if relu_out:
            o = jnp.maximum(o, 0.0)
        for c in range(ncy):
            out_ref[c] = o[:, c * 128:(c + 1) * 128]

    return pl.pallas_call(
        body,
        grid=(nb,),
        in_specs=[
            pl.BlockSpec((ncx, BM, 128), lambda b: (0, b, 0)),
            pl.BlockSpec((2, ncx, BM, 128), lambda b: (0, 0, b, 0)),
            pl.BlockSpec((ncx * 128, H1), cst),
            pl.BlockSpec((1, H1), cst),
            pl.BlockSpec((H1, Dout), cst),
            pl.BlockSpec((1, Dout), cst),
        ],
        out_specs=pl.BlockSpec((ncy, BM, 128), lambda b: (0, b, 0)),
        out_shape=jax.ShapeDtypeStruct((ncy, NN, 128), jnp.float32),
    )(x_ch, agg, W1, b1.reshape(1, H1), W2, b2.reshape(1, Dout))


def _gin_mlp_final(x_ch, agg, W1, b1, W2, b2):
    ncx = x_ch.shape[0]
    H1 = W1.shape[1]
    Dout = W2.shape[1]
    nb = NN // BM
    cst = lambda b: (0, 0)

    def body(x_ref, agg_ref, W1_ref, b1_ref, W2_ref, b2_ref,
             out_ref, g_ref, gacc_ref):
        b = pl.program_id(0)
        W1v = W1_ref[...]
        acc = jnp.zeros((BM, H1), jnp.float32)
        for c in range(ncx):
            t = x_ref[c] + agg_ref[0, c] + agg_ref[1, c]
            acc = acc + jnp.dot(t, W1v[c * 128:(c + 1) * 128, :],
                                preferred_element_type=jnp.float32)
        h = jnp.maximum(acc + b1_ref[...], 0.0)
        o = jnp.dot(h, W2_ref[...], preferred_element_type=jnp.float32) + b2_ref[...]
        out_ref[...] = o

        @pl.when(b == 0)
        def _():
            gacc_ref[...] = jnp.zeros_like(gacc_ref)
        gacc_ref[...] += jnp.sum(o, axis=0, keepdims=True)

        @pl.when(b == nb - 1)
        def _():
            g_ref[...] = gacc_ref[...] * (1.0 / NN)

    return pl.pallas_call(
        body,
        grid=(nb,),
        in_specs=[
            pl.BlockSpec((ncx, BM, 128), lambda b: (0, b, 0)),
            pl.BlockSpec((2, ncx, BM, 128), lambda b: (0, 0, b, 0)),
            pl.BlockSpec((ncx * 128, H1), cst),
            pl.BlockSpec((1, H1), cst),
            pl.BlockSpec((H1, Dout), cst),
            pl.BlockSpec((1, Dout), cst),
        ],
        out_specs=[
            pl.BlockSpec((BM, Dout), lambda b: (b, 0)),
            pl.BlockSpec((1, Dout), cst),
        ],
        out_shape=[
            jax.ShapeDtypeStruct((NN, Dout), jnp.float32),
            jax.ShapeDtypeStruct((1, Dout), jnp.float32),
        ],
        scratch_shapes=[pltpu.VMEM((1, Dout), jnp.float32)],
    )(x_ch, agg, W1, b1.reshape(1, H1), W2, b2.reshape(1, Dout))


def _sc_edge(ne, src, dst):
    BE = 128
    per_w = E // 32
    nfull = per_w // BE
    tail = per_w - nfull * BE
    mesh = plsc.VectorSubcoreMesh(core_axis_name="c", subcore_axis_name="s")

    @functools.partial(
        pl.kernel,
        out_type=jax.ShapeDtypeStruct((E, 256), jnp.float32),
        mesh=mesh,
        scratch_types=[
            pltpu.VMEM((BE,), jnp.int32),
            pltpu.VMEM((BE,), jnp.int32),
            pltpu.VMEM((BE, 128), jnp.float32),
            pltpu.VMEM((BE, 128), jnp.float32),
            pltpu.VMEM((tail,), jnp.int32),
            pltpu.VMEM((tail,), jnp.int32),
            pltpu.VMEM((tail, 128), jnp.float32),
            pltpu.VMEM((tail, 128), jnp.float32),
            pltpu.SemaphoreType.DMA,
            pltpu.SemaphoreType.DMA,
        ],
    )
    def k(ne_hbm, src_hbm, dst_hbm, out_hbm,
          si_v, di_v, rs_v, rd_v, sit_v, dit_v, rst_v, rdt_v, sem1, sem2):
        cid = lax.axis_index("c")
        sid = lax.axis_index("s")
        base0 = (sid * 2 + cid) * per_w

        def blk(i, carry):
            eb = base0 + i * BE
            pltpu.sync_copy(src_hbm.at[pl.ds(eb, BE)], si_v)
            pltpu.sync_copy(dst_hbm.at[pl.ds(eb, BE)], di_v)
            c1 = pltpu.async_copy(ne_hbm.at[si_v], rs_v, sem1)
            c2 = pltpu.async_copy(ne_hbm.at[di_v], rd_v, sem2)
            c1.wait()
            c2.wait()
            pltpu.sync_copy(rs_v, out_hbm.at[pl.ds(eb, BE), pl.ds(0, 128)])
            pltpu.sync_copy(rd_v, out_hbm.at[pl.ds(eb, BE), pl.ds(128, 128)])
            return carry
        lax.fori_loop(0, nfull, blk, 0)

        ebt = base0 + nfull * BE
        pltpu.sync_copy(src_hbm.at[pl.ds(ebt, tail)], sit_v)
        pltpu.sync_copy(dst_hbm.at[pl.ds(ebt, tail)], dit_v)
        c1 = pltpu.async_copy(ne_hbm.at[sit_v], rst_v, sem1)
        c2 = pltpu.async_copy(ne_hbm.at[dit_v], rdt_v, sem2)
        c1.wait()
        c2.wait()
        pltpu.sync_copy(rst_v, out_hbm.at[pl.ds(ebt, tail), pl.ds(0, 128)])
        pltpu.sync_copy(rdt_v, out_hbm.at[pl.ds(ebt, tail), pl.ds(128, 128)])

    return k(ne, src, dst)


def kernel(task_state_scheduled, task_state_ready, task_lengths,
           vm_completion_times, vm_speeds, vm_energy_rates, params,
           task_assignments, compatibilities, task_dependencies):
    p = params
    task_x = jnp.stack([task_state_scheduled, task_state_ready, task_lengths], axis=-1)
    vm_x = jnp.stack([vm_completion_times, vm_speeds, vm_energy_rates], axis=-1)

    task_h = _encoder(task_x, p['task_W1'], p['task_b1'], p['task_g1'], p['task_be1'],
                      p['task_W2'], p['task_b2'], p['task_g2'], p['task_be2'],
                      p['task_W3'], p['task_b3'])
    vm_h = _encoder(vm_x, p['vm_W1'], p['vm_b1'], p['vm_g1'], p['vm_be1'],
                    p['vm_W2'], p['vm_b2'], p['vm_g2'], p['vm_be2'],
                    p['vm_W3'], p['vm_b3'])
    node_x = jnp.concatenate([task_h, vm_h], axis=0)

    src = jnp.concatenate([compatibilities[0], task_dependencies[0]])
    dst = jnp.concatenate([compatibilities[1] + NT, task_dependencies[1]])

    x1 = node_x.reshape(1, NN, 128)
    agg1 = _sc_agg(x1, src, dst)
    h1 = _gin_mlp(x1, agg1, p['gin_W1a'], p['gin_b1a'], p['gin_W1b'], p['gin_b1b'], True)
    agg2 = _sc_agg(h1, src, dst)
    h2 = _gin_mlp(h1, agg2, p['gin_W2a'], p['gin_b2a'], p['gin_W2b'], p['gin_b2b'], True)
    agg3 = _sc_agg(h2, src, dst)
    node_emb, graph_emb = _gin_mlp_final(h2, agg3, p['gin_W3a'], p['gin_b3a'],
                                         p['gin_W3b'], p['gin_b3b'])
    edge_emb = _sc_edge(node_emb, src, dst)
    return node_emb, edge_emb, graph_emb
